# baseline (speedup 1.0000x reference)
import dataclasses
import functools

import jax
import jax.numpy as jnp
from jax import lax
from jax.experimental import pallas as pl
from jax.experimental.pallas import tpu as pltpu
from jax.experimental.pallas import tpu_sc as plsc

_LANES = 16
_NWORKERS = 32
_NBUF = 4
_PAIR = 1
_UNROLL = 8


def _interp_kernel(B, L, C, T):
    CT = C // 8
    LT = L // 128
    TT = T // 128
    nslab = B * CT
    spw = nslab // _NWORKERS // _PAIR
    groups = T // _LANES
    slab = LT * 8 * 128
    oslab = TT * 8 * 128
    mesh = plsc.VectorSubcoreMesh(core_axis_name="c", subcore_axis_name="s")
    cp = pltpu.CompilerParams()
    if "needs_layout_passes" in pltpu.CompilerParams.__dataclass_fields__:
        cp = dataclasses.replace(cp, needs_layout_passes=False)
    if "use_tc_tiling_on_sc" in pltpu.CompilerParams.__dataclass_fields__:
        cp = dataclasses.replace(cp, use_tc_tiling_on_sc=False)

    @functools.partial(
        pl.kernel,
        out_type=jax.ShapeDtypeStruct((nslab // _PAIR, _PAIR * oslab),
                                      jnp.float32),
        mesh=mesh,
        compiler_params=cp,
        scratch_types=(
            [pltpu.VMEM((T,), jnp.float32),
             pltpu.VMEM((T,), jnp.int32),
             pltpu.VMEM((T,), jnp.int32),
             pltpu.VMEM((T,), jnp.float32)]
            + [pltpu.VMEM((_PAIR * slab,), jnp.float32)] * _NBUF
            + [pltpu.VMEM((_PAIR * oslab,), jnp.float32)] * _NBUF
            + [pltpu.SemaphoreType.DMA] * (2 * _NBUF)
        ),
    )
    def k(x_hbm, tq_hbm, y_hbm, tq_v, a0_v, a1_v, fr_v, *bufs):
        knots_b = bufs[:_NBUF]
        out_b = bufs[_NBUF:2 * _NBUF]
        gsems = bufs[2 * _NBUF:3 * _NBUF]
        osems = bufs[3 * _NBUF:]
        wid = lax.axis_index("s") * 2 + lax.axis_index("c")
        k0 = wid * spw
        pltpu.sync_copy(tq_hbm, tq_v)

        @pl.loop(0, groups)
        def _(g):
            sl = pl.ds(g * _LANES, _LANES)
            t = tq_v[sl]
            ti = jnp.minimum(jnp.maximum(t.astype(jnp.int32), 0), L - 2)
            ti1 = ti + 1
            fr_v[sl] = t - ti.astype(jnp.float32)
            a0_v[sl] = ((ti >> 7) << 10) | (ti & 127)
            a1_v[sl] = ((ti1 >> 7) << 10) | (ti1 & 127)

        def fire(kk, knots, gsem):
            pltpu.async_copy(x_hbm.at[k0 + kk], knots, gsem)

        def drain(kk, knots, gsem):
            pltpu.make_async_copy(x_hbm.at[k0 + kk], knots, gsem).wait()

        def compute(kk, knots, outv, osem):
            @plsc.parallel_loop(0, groups, unroll=_UNROLL)
            def _(g):
                sl = pl.ds(g * _LANES, _LANES)
                a0 = a0_v[sl]
                a1 = a1_v[sl]
                f = fr_v[sl]
                obase = (g // 8) * 1024 + (g % 8) * _LANES
                for p in range(_PAIR):
                    for ci in range(8):
                        off = p * slab + ci * 128
                        gp = plsc.load_gather(knots, [a0 + off])
                        gn = plsc.load_gather(knots, [a1 + off])
                        outv[pl.ds(p * oslab + obase + ci * 128, _LANES)] = (
                            gp + f * (gn - gp))
            pltpu.async_copy(outv, y_hbm.at[k0 + kk], osem)

        def wait_out(kk, outv, osem):
            pltpu.make_async_copy(outv, y_hbm.at[k0 + kk], osem).wait()

        for s in range(_NBUF):
            fire(s, knots_b[s], gsems[s])

        @pl.loop(0, spw, step=_NBUF)
        def _(kk):
            for s in range(_NBUF):
                drain(kk + s, knots_b[s], gsems[s])

                @pl.when(kk >= _NBUF)
                def _():
                    wait_out(kk + s - _NBUF, out_b[s], osems[s])

                compute(kk + s, knots_b[s], out_b[s], osems[s])

                @pl.when(kk + s + _NBUF < spw)
                def _():
                    fire(kk + s + _NBUF, knots_b[s], gsems[s])

        for s in range(_NBUF):
            wait_out(spw - _NBUF + s, out_b[s], osems[s])

    return k


def kernel(coeffs, t_query):
    B, L, C = coeffs.shape
    T = t_query.shape[0]
    x = (coeffs.reshape(B, L // 128, 128, C // 8, 8)
         .transpose(0, 3, 1, 4, 2)
         .reshape(B * (C // 8) // _PAIR, _PAIR * (L // 128) * 8 * 128))
    y = _interp_kernel(B, L, C, T)(x, t_query)
    out = (y.reshape(B, C // 8, T // 128, 8, 128)
           .transpose(0, 2, 4, 1, 3)
           .reshape(B, T, C))
    return out


# speedup vs baseline: 1.1223x; 1.1223x over previous
import dataclasses
import functools

import jax
import jax.numpy as jnp
from jax import lax
from jax.experimental import pallas as pl
from jax.experimental.pallas import tpu as pltpu
from jax.experimental.pallas import tpu_sc as plsc

_LANES = 16
_NWORKERS = 32
_NBUF = 4
_PAIR = 1
_UNROLL = 4


def _interp_kernel(B, L, C, T):
    CT = C // 8
    LT = L // 128
    TT = T // 128
    nslab = B * CT
    spw = nslab // _NWORKERS // _PAIR
    groups = T // _LANES
    slab = LT * 8 * 128
    oslab = TT * 8 * 128
    mesh = plsc.VectorSubcoreMesh(core_axis_name="c", subcore_axis_name="s")
    cp = pltpu.CompilerParams()
    if "needs_layout_passes" in pltpu.CompilerParams.__dataclass_fields__:
        cp = dataclasses.replace(cp, needs_layout_passes=False)
    if "use_tc_tiling_on_sc" in pltpu.CompilerParams.__dataclass_fields__:
        cp = dataclasses.replace(cp, use_tc_tiling_on_sc=False)

    @functools.partial(
        pl.kernel,
        out_type=jax.ShapeDtypeStruct((nslab // _PAIR, _PAIR * oslab),
                                      jnp.float32),
        mesh=mesh,
        compiler_params=cp,
        scratch_types=(
            [pltpu.VMEM((T,), jnp.float32),
             pltpu.VMEM((T,), jnp.int32),
             pltpu.VMEM((T,), jnp.int32),
             pltpu.VMEM((T,), jnp.float32)]
            + [pltpu.VMEM((_PAIR * slab,), jnp.float32)] * _NBUF
            + [pltpu.VMEM((_PAIR * oslab,), jnp.float32)] * _NBUF
            + [pltpu.SemaphoreType.DMA] * (2 * _NBUF)
        ),
    )
    def k(x_hbm, tq_hbm, y_hbm, tq_v, a0_v, a1_v, fr_v, *bufs):
        knots_b = bufs[:_NBUF]
        out_b = bufs[_NBUF:2 * _NBUF]
        gsems = bufs[2 * _NBUF:3 * _NBUF]
        osems = bufs[3 * _NBUF:]
        wid = lax.axis_index("s") * 2 + lax.axis_index("c")
        k0 = wid * spw
        pltpu.sync_copy(tq_hbm, tq_v)

        @pl.loop(0, groups)
        def _(g):
            sl = pl.ds(g * _LANES, _LANES)
            t = tq_v[sl]
            ti = jnp.minimum(jnp.maximum(t.astype(jnp.int32), 0), L - 2)
            ti1 = ti + 1
            fr_v[sl] = t - ti.astype(jnp.float32)
            a0_v[sl] = ((ti >> 7) << 10) | (ti & 127)
            a1_v[sl] = ((ti1 >> 7) << 10) | (ti1 & 127)

        def fire(kk, knots, gsem):
            pltpu.async_copy(x_hbm.at[k0 + kk], knots, gsem)

        def drain(kk, knots, gsem):
            pltpu.make_async_copy(x_hbm.at[k0 + kk], knots, gsem).wait()

        def compute(kk, knots, outv, osem):
            @plsc.parallel_loop(0, groups, unroll=_UNROLL)
            def _(g):
                sl = pl.ds(g * _LANES, _LANES)
                a0 = a0_v[sl]
                a1 = a1_v[sl]
                f = fr_v[sl]
                obase = (g // 8) * 1024 + (g % 8) * _LANES
                for p in range(_PAIR):
                    for ci in range(8):
                        off = p * slab + ci * 128
                        gp = plsc.load_gather(knots, [a0 + off])
                        gn = plsc.load_gather(knots, [a1 + off])
                        outv[pl.ds(p * oslab + obase + ci * 128, _LANES)] = (
                            gp + f * (gn - gp))
            pltpu.async_copy(outv, y_hbm.at[k0 + kk], osem)

        def wait_out(kk, outv, osem):
            pltpu.make_async_copy(outv, y_hbm.at[k0 + kk], osem).wait()

        for s in range(_NBUF):
            fire(s, knots_b[s], gsems[s])

        @pl.loop(0, spw, step=_NBUF)
        def _(kk):
            for s in range(_NBUF):
                drain(kk + s, knots_b[s], gsems[s])

                @pl.when(kk >= _NBUF)
                def _():
                    wait_out(kk + s - _NBUF, out_b[s], osems[s])

                compute(kk + s, knots_b[s], out_b[s], osems[s])

                @pl.when(kk + s + _NBUF < spw)
                def _():
                    fire(kk + s + _NBUF, knots_b[s], gsems[s])

        for s in range(_NBUF):
            wait_out(spw - _NBUF + s, out_b[s], osems[s])

    return k


def kernel(coeffs, t_query):
    B, L, C = coeffs.shape
    T = t_query.shape[0]
    x = (coeffs.reshape(B, L // 128, 128, C // 8, 8)
         .transpose(0, 3, 1, 4, 2)
         .reshape(B * (C // 8) // _PAIR, _PAIR * (L // 128) * 8 * 128))
    y = _interp_kernel(B, L, C, T)(x, t_query)
    out = (y.reshape(B, C // 8, T // 128, 8, 128)
           .transpose(0, 2, 4, 1, 3)
           .reshape(B, T, C))
    return out
